# baseline (speedup 1.0000x reference)
import functools

import jax
import jax.numpy as jnp
from jax import lax
from jax.experimental import pallas as pl
from jax.experimental.pallas import tpu as pltpu
from jax.experimental.pallas import tpu_sc as plsc

_BATCH = 4096
_NF = 26
_NV = 100
_D = _NF * _NV
_NC = 2
_NS = 16
_CROWS = 2 * _NV
_CB = 128
_NCH = _NF // 2


def _sc_body(idx_hbm, out_hbm, buf0, buf1, idx_v, sem0, sem1):
    wid = lax.axis_index("s") * _NC + lax.axis_index("c")
    c0 = wid * _CB

    zeros = jnp.zeros((16,), jnp.int32)
    ones = jnp.ones((16,), jnp.int32)
    lane = lax.iota(jnp.int32, 16)

    pltpu.sync_copy(idx_hbm.at[:, pl.ds(c0, _CB)], idx_v)

    def zero_body(i, carry):
        for rr in range(4):
            for s in range(8):
                buf0[i * 4 + rr, pl.ds(s * 16, 16)] = zeros
                buf1[i * 4 + rr, pl.ds(s * 16, 16)] = zeros
        return carry

    lax.fori_loop(0, _CROWS // 4, zero_body, 0)

    bufs = (buf0, buf1)
    sems = (sem0, sem1)

    def win(p):
        r = pl.multiple_of(p * _CROWS, _CROWS)
        return out_hbm.at[pl.ds(r, _CROWS), pl.ds(c0, _CB)]

    def scat(buf, p, x):
        for fi in range(2):
            for g in range(8):
                vals = idx_v[2 * p + fi, pl.ds(g * 16, 16)]
                plsc.store_scatter(
                    buf, [vals + (fi * _NV), lane + (g * 16)], x)

    def step(buf, sem, p):
        pltpu.make_async_copy(buf, win(p - 2), sem).wait()
        scat(buf, p - 2, zeros)
        scat(buf, p, ones)
        pltpu.async_copy(buf, win(p), sem)

    scat(buf0, 0, ones)
    pltpu.async_copy(buf0, win(0), sem0)
    scat(buf1, 1, ones)
    pltpu.async_copy(buf1, win(1), sem1)

    def pair_body(j, carry):
        step(buf0, sem0, 2 * j + 2)
        step(buf1, sem1, 2 * j + 3)
        return carry

    lax.fori_loop(0, (_NCH - 3) // 2, pair_body, 0)

    step(buf0, sem0, _NCH - 1)
    pltpu.make_async_copy(buf1, win(_NCH - 2), sem1).wait()
    pltpu.make_async_copy(buf0, win(_NCH - 1), sem0).wait()


@functools.partial(
    pl.kernel,
    out_type=jax.ShapeDtypeStruct((_D, _BATCH), jnp.int32),
    mesh=plsc.VectorSubcoreMesh(
        core_axis_name="c", subcore_axis_name="s",
        num_cores=_NC, num_subcores=_NS,
    ),
    scratch_types=[
        pltpu.VMEM((_CROWS, _CB), jnp.int32),
        pltpu.VMEM((_CROWS, _CB), jnp.int32),
        pltpu.VMEM((_NF, _CB), jnp.int32),
        pltpu.SemaphoreType.DMA,
        pltpu.SemaphoreType.DMA,
    ],
    compiler_params=pltpu.CompilerParams(
        needs_layout_passes=False,
        disable_bounds_checks=True,
        disable_semaphore_checks=True,
    ),
)
def _sc_multi_one_hot(idx_hbm, out_hbm, buf0, buf1, idx_v, sem0, sem1):
    _sc_body(idx_hbm, out_hbm, buf0, buf1, idx_v, sem0, sem1)


@jax.jit
def kernel(index_list):
    return _sc_multi_one_hot(index_list.T).T


# speedup vs baseline: 1.0028x; 1.0028x over previous
import functools

import jax
import jax.numpy as jnp
from jax import lax
from jax.experimental import pallas as pl
from jax.experimental.pallas import tpu as pltpu
from jax.experimental.pallas import tpu_sc as plsc

_BATCH = 4096
_NF = 26
_NV = 100
_D = _NF * _NV
_NC = 2
_NS = 16
_CROWS = 2 * _NV
_CB = 128
_NCH = _NF // 2


def _sc_body(idx_hbm, out_hbm, buf0, buf1, idx_v, sem0, sem1):
    wid = lax.axis_index("s") * _NC + lax.axis_index("c")
    c0 = wid * _CB

    zeros = jnp.zeros((16,), jnp.int32)
    ones = jnp.ones((16,), jnp.int32)
    lane = lax.iota(jnp.int32, 16)

    pltpu.sync_copy(idx_hbm.at[:, pl.ds(c0, _CB)], idx_v)

    def zero_body(i, carry):
        for rr in range(4):
            for s in range(8):
                buf0[i * 4 + rr, pl.ds(s * 16, 16)] = zeros
                buf1[i * 4 + rr, pl.ds(s * 16, 16)] = zeros
        return carry

    lax.fori_loop(0, _CROWS // 4, zero_body, 0)

    bufs = (buf0, buf1)
    sems = (sem0, sem1)

    def win(p):
        r = pl.multiple_of(p * _CROWS, _CROWS)
        return out_hbm.at[pl.ds(r, _CROWS), pl.ds(c0, _CB)]

    def scat(buf, p, x):
        for fi in range(2):
            for g in range(8):
                vals = idx_v[2 * p + fi, pl.ds(g * 16, 16)]
                plsc.store_scatter(
                    buf, [vals + (fi * _NV), lane + (g * 16)], x)

    def step(buf, sem, p):
        pltpu.make_async_copy(buf, win(p - 2), sem).wait()
        scat(buf, p - 2, zeros)
        scat(buf, p, ones)
        pltpu.async_copy(buf, win(p), sem)

    scat(buf0, 0, ones)
    pltpu.async_copy(buf0, win(0), sem0)
    scat(buf1, 1, ones)
    pltpu.async_copy(buf1, win(1), sem1)

    def pair_body(j, carry):
        step(buf0, sem0, 2 * j + 2)
        step(buf1, sem1, 2 * j + 3)
        return carry

    lax.fori_loop(0, (_NCH - 3) // 2, pair_body, 0)

    step(buf0, sem0, _NCH - 1)
    pltpu.make_async_copy(buf1, win(_NCH - 2), sem1).wait()
    pltpu.make_async_copy(buf0, win(_NCH - 1), sem0).wait()


@functools.partial(
    pl.kernel,
    out_type=jax.ShapeDtypeStruct((_D, _BATCH), jnp.int32),
    mesh=plsc.VectorSubcoreMesh(
        core_axis_name="c", subcore_axis_name="s",
        num_cores=_NC, num_subcores=_NS,
    ),
    scratch_types=[
        pltpu.VMEM((_CROWS, _CB), jnp.int32),
        pltpu.VMEM((_CROWS, _CB), jnp.int32),
        pltpu.VMEM((_NF, _CB), jnp.int32),
        pltpu.SemaphoreType.DMA,
        pltpu.SemaphoreType.DMA,
    ],
    compiler_params=pltpu.CompilerParams(needs_layout_passes=False),
)
def _sc_multi_one_hot(idx_hbm, out_hbm, buf0, buf1, idx_v, sem0, sem1):
    _sc_body(idx_hbm, out_hbm, buf0, buf1, idx_v, sem0, sem1)


@jax.jit
def kernel(index_list):
    return _sc_multi_one_hot(index_list.T).T
